# baseline (speedup 1.0000x reference)
import functools

import jax
import jax.numpy as jnp
from jax import lax
from jax.experimental import pallas as pl
from jax.experimental.pallas import tpu as pltpu
from jax.experimental.pallas import tpu_sc as plsc


_INFO = plsc.get_sparse_core_info()
_NC = _INFO.num_cores
_NS = _INFO.num_subcores
_NW = _NC * _NS
_CH = 128


@functools.partial(jax.jit, static_argnums=(2, 3))
def _sc_gather(idx_flat, table, nch, hidden):
    n_rows = _NW * nch * _CH
    mesh = plsc.VectorSubcoreMesh(core_axis_name="c", subcore_axis_name="s")

    nbuf = 3
    vocab = table.shape[0]

    @functools.partial(
        pl.kernel,
        out_type=jax.ShapeDtypeStruct((n_rows // _CH, _CH, hidden), jnp.float32),
        mesh=mesh,
        scratch_types=[
            pltpu.VMEM((nch * _CH,), jnp.int32),
            [pltpu.VMEM((1, _CH, hidden), jnp.float32)] * nbuf,
            pltpu.VMEM_SHARED((vocab, hidden), jnp.float32),
            [pltpu.SemaphoreType.DMA] * nbuf,
            [pltpu.SemaphoreType.DMA] * nbuf,
        ],
    )
    def body(idx_hbm, table_hbm, out_hbm, idx_v, bufs, tab_sh, gsems, ssems):
        wid = lax.axis_index("s") * _NC + lax.axis_index("c")
        base = wid * nch
        sid = lax.axis_index("s")

        @pl.when(sid == 0)
        def _stage_table():
            pltpu.sync_copy(table_hbm, tab_sh)

        pltpu.sync_copy(idx_hbm.at[pl.ds(wid * nch * _CH, nch * _CH)], idx_v)
        plsc.subcore_barrier()

        def gather(c, b):
            return pltpu.async_copy(
                tab_sh.at[idx_v.at[pl.ds(c * _CH, _CH)]], bufs[b].at[0], gsems[b]
            )

        gathers = [None] * nbuf
        stores = [None] * nbuf
        for c in range(min(nbuf, nch)):
            gathers[c] = gather(c, c)
        for c in range(nch):
            b = c % nbuf
            gathers[b].wait()
            stores[b] = pltpu.async_copy(
                bufs[b], out_hbm.at[pl.ds(base + c, 1)], ssems[b]
            )
            nxt = c + nbuf
            if nxt < nch:
                stores[b].wait()
                gathers[b] = gather(nxt, b)
                stores[b] = None
        for s in stores:
            if s is not None:
                s.wait()

    return body(idx_flat, table)


_BLK = 2048


def _tc_body(ids_ref, tab_ref, out_ref):
    ids = ids_ref[0, 0, :]
    onehot = (ids[:, None] == lax.broadcasted_iota(jnp.int32, (_BLK, tab_ref.shape[0]), 1)).astype(jnp.float32)
    out_ref[...] = jnp.dot(onehot, tab_ref[...], preferred_element_type=jnp.float32)


@jax.jit
def _tc_gather(idx_flat, table):
    n = idx_flat.shape[0]
    vocab, hidden = table.shape
    nblk = n // _BLK
    ids3 = idx_flat.reshape(nblk, 1, _BLK)
    return pl.pallas_call(
        _tc_body,
        grid=(nblk,),
        in_specs=[
            pl.BlockSpec((1, 1, _BLK), lambda i: (i, 0, 0)),
            pl.BlockSpec((vocab, hidden), lambda i: (0, 0)),
        ],
        out_specs=pl.BlockSpec((_BLK, hidden), lambda i: (i, 0)),
        out_shape=jax.ShapeDtypeStruct((n, hidden), jnp.float32),
    )(ids3, table)


def kernel(input_ids, attention_mask, embed_weight):
    del attention_mask
    batch, seq = input_ids.shape
    vocab, hidden = embed_weight.shape
    half = batch // 2
    n_sc = half * seq
    nch = n_sc // (_NW * _CH)
    ids = input_ids.astype(jnp.int32)
    table = embed_weight.astype(jnp.float32)
    sc_out = _sc_gather(ids[:half].reshape(-1), table, nch, hidden)
    tc_out = _tc_gather(ids[half:].reshape(-1), table)
    return jnp.concatenate(
        [sc_out.reshape(half, seq, hidden), tc_out.reshape(batch - half, seq, hidden)], axis=0
    )


# speedup vs baseline: 1.3565x; 1.3565x over previous
import functools

import jax
import jax.numpy as jnp
from jax import lax
from jax.experimental import pallas as pl
from jax.experimental.pallas import tpu as pltpu
from jax.experimental.pallas import tpu_sc as plsc


_INFO = plsc.get_sparse_core_info()
_NC = _INFO.num_cores
_NS = _INFO.num_subcores
_NW = _NC * _NS
_CH = 128


@functools.partial(jax.jit, static_argnums=(2, 3))
def _sc_gather(idx_flat, table, nch, hidden):
    n_rows = _NW * nch * _CH
    mesh = plsc.VectorSubcoreMesh(core_axis_name="c", subcore_axis_name="s")

    nbuf = 4
    vocab = table.shape[0]

    @functools.partial(
        pl.kernel,
        out_type=jax.ShapeDtypeStruct((n_rows // _CH, _CH, hidden), jnp.float32),
        mesh=mesh,
        scratch_types=[
            pltpu.VMEM((nch * _CH,), jnp.int32),
            [pltpu.VMEM((1, _CH, hidden), jnp.float32)] * nbuf,
            pltpu.VMEM_SHARED((vocab, hidden), jnp.float32),
            [pltpu.SemaphoreType.DMA] * nbuf,
            [pltpu.SemaphoreType.DMA] * nbuf,
        ],
    )
    def body(idx_hbm, table_hbm, out_hbm, idx_v, bufs, tab_sh, gsems, ssems):
        wid = lax.axis_index("s") * _NC + lax.axis_index("c")
        base = wid * nch
        sid = lax.axis_index("s")

        @pl.when(sid == 0)
        def _stage_table():
            pltpu.sync_copy(table_hbm, tab_sh)

        pltpu.sync_copy(idx_hbm.at[pl.ds(wid * nch * _CH, nch * _CH)], idx_v)
        plsc.subcore_barrier()

        def gather(c, b):
            return pltpu.async_copy(
                tab_sh.at[idx_v.at[pl.ds(c * _CH, _CH)]], bufs[b].at[0], gsems[b]
            )

        gathers = [None] * nbuf
        stores = [None] * nbuf
        for c in range(min(nbuf, nch)):
            gathers[c] = gather(c, c)
        for c in range(nch):
            b = c % nbuf
            gathers[b].wait()
            stores[b] = pltpu.async_copy(
                bufs[b], out_hbm.at[pl.ds(base + c, 1)], ssems[b]
            )
            nxt = c + nbuf
            if nxt < nch:
                stores[b].wait()
                gathers[b] = gather(nxt, b)
                stores[b] = None
        for s in stores:
            if s is not None:
                s.wait()

    return body(idx_flat, table)


def kernel(input_ids, attention_mask, embed_weight):
    del attention_mask
    batch, seq = input_ids.shape
    vocab, hidden = embed_weight.shape
    n_rows = batch * seq
    nch = n_rows // (_NW * _CH)
    ids = input_ids.reshape(-1).astype(jnp.int32)
    table = embed_weight.astype(jnp.float32)
    out = _sc_gather(ids, table, nch, hidden)
    return out.reshape(batch, seq, hidden)


# speedup vs baseline: 1.3752x; 1.0138x over previous
import functools

import jax
import jax.numpy as jnp
from jax import lax
from jax.experimental import pallas as pl
from jax.experimental.pallas import tpu as pltpu
from jax.experimental.pallas import tpu_sc as plsc


_INFO = plsc.get_sparse_core_info()
_NC = _INFO.num_cores
_NS = _INFO.num_subcores
_NW = _NC * _NS
_CH = 128


@functools.partial(jax.jit, static_argnums=(2, 3))
def _sc_gather(ids2d, table, nch, hidden):
    batch, seq = ids2d.shape
    n_rows = batch * seq
    wpb = _NW // batch
    mesh = plsc.VectorSubcoreMesh(core_axis_name="c", subcore_axis_name="s")

    nbuf = 3
    vocab = table.shape[0]

    @functools.partial(
        pl.kernel,
        out_type=jax.ShapeDtypeStruct((n_rows // _CH, _CH, hidden), jnp.float32),
        mesh=mesh,
        scratch_types=[
            pltpu.VMEM((nch * _CH,), jnp.int32),
            [pltpu.VMEM((1, _CH, hidden), jnp.float32)] * nbuf,
            pltpu.VMEM_SHARED((vocab, hidden), jnp.float32),
            [pltpu.SemaphoreType.DMA] * nbuf,
            [pltpu.SemaphoreType.DMA] * nbuf,
        ],
    )
    def body(idx_hbm, table_hbm, out_hbm, idx_v, bufs, tab_sh, gsems, ssems):
        wid = lax.axis_index("s") * _NC + lax.axis_index("c")
        base = wid * nch
        row = wid // wpb
        col = (wid % wpb) * (nch * _CH)
        sid = lax.axis_index("s")

        @pl.when(sid == 0)
        def _stage_table():
            pltpu.sync_copy(table_hbm, tab_sh)

        pltpu.sync_copy(idx_hbm.at[row, pl.ds(col, nch * _CH)], idx_v)
        plsc.subcore_barrier()

        def gather(c, b):
            return pltpu.async_copy(
                tab_sh.at[idx_v.at[pl.ds(c * _CH, _CH)]], bufs[b].at[0], gsems[b]
            )

        gathers = [None] * nbuf
        stores = [None] * nbuf
        for c in range(min(nbuf, nch)):
            gathers[c] = gather(c, c)
        for c in range(nch):
            b = c % nbuf
            gathers[b].wait()
            stores[b] = pltpu.async_copy(
                bufs[b], out_hbm.at[pl.ds(base + c, 1)], ssems[b]
            )
            nxt = c + nbuf
            if nxt < nch:
                stores[b].wait()
                gathers[b] = gather(nxt, b)
                stores[b] = None
        for s in stores:
            if s is not None:
                s.wait()

    return body(ids2d, table)


def kernel(input_ids, attention_mask, embed_weight):
    del attention_mask
    batch, seq = input_ids.shape
    vocab, hidden = embed_weight.shape
    n_rows = batch * seq
    nch = n_rows // (_NW * _CH)
    ids = input_ids.astype(jnp.int32)
    table = embed_weight.astype(jnp.float32)
    out = _sc_gather(ids, table, nch, hidden)
    return out.reshape(batch, seq, hidden)


# speedup vs baseline: 1.3795x; 1.0032x over previous
import functools

import jax
import jax.numpy as jnp
from jax import lax
from jax.experimental import pallas as pl
from jax.experimental.pallas import tpu as pltpu
from jax.experimental.pallas import tpu_sc as plsc


_INFO = plsc.get_sparse_core_info()
_NC = _INFO.num_cores
_NS = _INFO.num_subcores
_NW = _NC * _NS
_CH = 128


@functools.partial(jax.jit, static_argnums=(2, 3))
def _sc_gather(idx_flat, table, nch, hidden):
    n_rows = _NW * nch * _CH
    mesh = plsc.VectorSubcoreMesh(core_axis_name="c", subcore_axis_name="s")

    nbuf = 3
    vocab = table.shape[0]

    @functools.partial(
        pl.kernel,
        out_type=jax.ShapeDtypeStruct((n_rows // _CH, _CH, hidden), jnp.float32),
        mesh=mesh,
        scratch_types=[
            pltpu.VMEM((nch * _CH,), jnp.int32),
            [pltpu.VMEM((1, _CH, hidden), jnp.float32)] * nbuf,
            pltpu.VMEM_SHARED((vocab, hidden), jnp.float32),
            [pltpu.SemaphoreType.DMA] * nbuf,
            [pltpu.SemaphoreType.DMA] * nbuf,
        ],
    )
    def body(idx_hbm, table_hbm, out_hbm, idx_v, bufs, tab_sh, gsems, ssems):
        wid = lax.axis_index("s") * _NC + lax.axis_index("c")
        base = wid * nch
        sid = lax.axis_index("s")

        @pl.when(sid == 0)
        def _stage_table():
            pltpu.sync_copy(table_hbm, tab_sh)

        pltpu.sync_copy(idx_hbm.at[pl.ds(wid * nch * _CH, nch * _CH)], idx_v)
        plsc.subcore_barrier()

        def gather(c, b):
            return pltpu.async_copy(
                tab_sh.at[idx_v.at[pl.ds(c * _CH, _CH)]], bufs[b].at[0], gsems[b]
            )

        gathers = [None] * nbuf
        stores = [None] * nbuf
        for c in range(min(nbuf, nch)):
            gathers[c] = gather(c, c)
        for c in range(nch):
            b = c % nbuf
            gathers[b].wait()
            stores[b] = pltpu.async_copy(
                bufs[b], out_hbm.at[pl.ds(base + c, 1)], ssems[b]
            )
            nxt = c + nbuf
            if nxt < nch:
                stores[b].wait()
                gathers[b] = gather(nxt, b)
                stores[b] = None
        for s in stores:
            if s is not None:
                s.wait()

    return body(idx_flat, table)


def kernel(input_ids, attention_mask, embed_weight):
    del attention_mask
    batch, seq = input_ids.shape
    vocab, hidden = embed_weight.shape
    n_rows = batch * seq
    nch = n_rows // (_NW * _CH)
    ids = input_ids.reshape(-1).astype(jnp.int32)
    table = embed_weight.astype(jnp.float32)
    out = _sc_gather(ids, table, nch, hidden)
    return out.reshape(batch, seq, hidden)
